# Initial kernel scaffold; baseline (speedup 1.0000x reference)
import jax
import jax.numpy as jnp
from jax.experimental import pallas as pl


def kernel(x, edge_index, W_self, W_neigh, b, W_e2d):
    raise NotImplementedError("write your pallas kernel here")


# speedup vs baseline: 3.6677x; 3.6677x over previous
import jax
import jax.numpy as jnp
from jax import lax
from jax.experimental import pallas as pl
from jax.experimental.pallas import tpu as pltpu
from jax.experimental.pallas import tpu_sc as plsc

N = 10000
E = 160000
D = 256
H = 256
HALF = 128
NC = 2
NS = 16
EPW = E // NS
CH = 128
NCHUNK = -(-EPW // CH)
EPWP = NCHUNK * CH
TRASH = N
DROW = 80
NDEG0 = 10112
NACC = 10240
STRIPE = NACC // NS
NV16 = EPWP // 16
BLK = 1000


def _sc_body(xl, xr, src_i, dst_i, idn, zr8, out,
             src_v, dst_v, buf, deg_v, idn_v, acc, sem):
    c = lax.axis_index("c")
    s = lax.axis_index("s")
    pltpu.sync_copy(src_i.at[s], src_v)
    pltpu.sync_copy(dst_i.at[s], dst_v)

    def zstep(t, carry):
        pltpu.sync_copy(zr8, acc.at[pl.ds(s * STRIPE + t * 8, 8)])
        return carry

    lax.fori_loop(0, STRIPE // 8, zstep, 0)

    @pl.when(c == 0)
    def _deg():
        pltpu.sync_copy(idn, idn_v)

        def dzero(t, carry):
            deg_v[t >> 3, pl.ds((t & 7) * 16, 16)] = jnp.zeros(
                (16,), jnp.float32)
            return carry

        lax.fori_loop(0, DROW * 8, dzero, 0)

        def dstep(t, carry):
            idx = dst_v[t >> 3, pl.ds((t & 7) * 16, 16)]
            plsc.addupdate_scatter(
                deg_v,
                [lax.shift_right_logical(idx, 7), lax.bitwise_and(idx, 127)],
                jnp.ones((16,), jnp.float32))
            return carry

        lax.fori_loop(0, NV16, dstep, 0)

    plsc.subcore_barrier()

    @pl.when(c == 0)
    def _degred():
        pltpu.sync_copy(deg_v, acc.at[idn_v.at[0]], add=True)

    def make_loop(table):
        def step(j, carry):
            pltpu.async_copy(table.at[src_v.at[j]], buf, sem).wait()
            pltpu.sync_copy(buf, acc.at[dst_v.at[j]], add=True)
            return carry
        return step

    @pl.when(c == 0)
    def _edges_l():
        lax.fori_loop(0, NCHUNK, make_loop(xl), 0)

    @pl.when(c == 1)
    def _edges_r():
        lax.fori_loop(0, NCHUNK, make_loop(xr), 0)

    plsc.subcore_barrier()
    pltpu.sync_copy(acc.at[pl.ds(s * STRIPE, STRIPE)],
                    out.at[c, pl.ds(s * STRIPE, STRIPE)])


_sc_agg = pl.kernel(
    _sc_body,
    mesh=plsc.VectorSubcoreMesh(core_axis_name="c", subcore_axis_name="s"),
    compiler_params=pltpu.CompilerParams(needs_layout_passes=False),
    out_type=jax.ShapeDtypeStruct((NC, NACC, HALF), jnp.float32),
    scratch_types=[
        pltpu.VMEM((NCHUNK, CH), jnp.int32),
        pltpu.VMEM((NCHUNK, CH), jnp.int32),
        pltpu.VMEM((CH, HALF), jnp.float32),
        pltpu.VMEM((DROW, HALF), jnp.float32),
        pltpu.VMEM((1, DROW), jnp.int32),
        pltpu.VMEM_SHARED((NACC, HALF), jnp.float32),
        pltpu.SemaphoreType.DMA,
    ],
)


def _tc_body(x_ref, al_ref, ar_ref, dg_ref, ws_ref, wn1_ref, wn2_ref,
             b_ref, we_ref, out_ref):
    u = (jnp.dot(al_ref[...], wn1_ref[...], preferred_element_type=jnp.float32)
         + jnp.dot(ar_ref[...], wn2_ref[...],
                   preferred_element_type=jnp.float32))
    h = (jnp.dot(x_ref[...], ws_ref[...], preferred_element_type=jnp.float32)
         + u / jnp.maximum(dg_ref[...], 1.0) + b_ref[...])
    h = jnp.maximum(h, 0.0)
    out_ref[...] = jnp.dot(h, we_ref[...], preferred_element_type=jnp.float32)


_tc_dense = pl.pallas_call(
    _tc_body,
    grid=(N // BLK,),
    in_specs=[
        pl.BlockSpec((BLK, D), lambda i: (i, 0)),
        pl.BlockSpec((BLK, HALF), lambda i: (i, 0)),
        pl.BlockSpec((BLK, HALF), lambda i: (i, 0)),
        pl.BlockSpec((BLK, 1), lambda i: (i, 0)),
        pl.BlockSpec((D, H), lambda i: (0, 0)),
        pl.BlockSpec((HALF, H), lambda i: (0, 0)),
        pl.BlockSpec((HALF, H), lambda i: (0, 0)),
        pl.BlockSpec((1, H), lambda i: (0, 0)),
        pl.BlockSpec((H, H), lambda i: (0, 0)),
    ],
    out_specs=pl.BlockSpec((BLK, H), lambda i: (i, 0)),
    out_shape=jax.ShapeDtypeStruct((N, H), jnp.float32),
)


def kernel(x, edge_index, W_self, W_neigh, b, W_e2d):
    src = edge_index[0].astype(jnp.int32)
    dst = edge_index[1].astype(jnp.int32)

    xl = x[:, :HALF]
    xr = x[:, HALF:]

    src_i = jnp.pad(src.reshape(NS, EPW),
                    ((0, 0), (0, EPWP - EPW))).reshape(NS, NCHUNK, CH)
    dst_i = jnp.pad(dst.reshape(NS, EPW), ((0, 0), (0, EPWP - EPW)),
                    constant_values=TRASH).reshape(NS, NCHUNK, CH)
    idn = (jnp.arange(DROW, dtype=jnp.int32) + NDEG0)[None, :]
    zr8 = jnp.zeros((8, HALF), jnp.float32)

    accs = _sc_agg(xl, xr, src_i, dst_i, idn, zr8)

    deg = accs[0, NDEG0:NDEG0 + DROW].reshape(DROW * HALF)[:N, None]
    return _tc_dense(x, accs[0, :N], accs[1, :N], deg, W_self,
                     W_neigh[:HALF], W_neigh[HALF:], b[None, :], W_e2d)
